# baseline (speedup 1.0000x reference)
import functools

import jax
import jax.numpy as jnp
import numpy as np
from jax.experimental import pallas as pl

_INDEX_SHIFT = np.array(
    [[0, 0], [-1, 0], [1, 0], [0, 1], [-1, 1], [1, 1], [0, -1], [-1, -1], [1, -1]],
    dtype=np.int32,
)
_H, _W = 496, 432
_C = 128
_N = 20000


def _qkv_body(lf_ref, rf_ref, wq1, wk1, wv1, wq2, wk2, wv2,
              q1, k1, v1, q2, k2, v2):
    lf = lf_ref[...]
    rf = rf_ref[...]
    q1[...] = jnp.dot(lf, wq1[...], preferred_element_type=jnp.float32)
    k1[...] = jnp.dot(rf, wk1[...], preferred_element_type=jnp.float32)
    v1[...] = jnp.dot(rf, wv1[...], preferred_element_type=jnp.float32)
    q2[...] = jnp.dot(rf, wq2[...], preferred_element_type=jnp.float32)
    k2[...] = jnp.dot(lf, wk2[...], preferred_element_type=jnp.float32)
    v2[...] = jnp.dot(lf, wv2[...], preferred_element_type=jnp.float32)


def _qkv(lf, rf, Wq1, Wk1, Wv1, Wq2, Wk2, Wv2):
    n = lf.shape[0]
    blk = 2000
    grid = (n // blk,)
    row_spec = pl.BlockSpec((blk, _C), lambda i: (i, 0))
    w_spec = pl.BlockSpec((_C, _C), lambda i: (0, 0))
    out_sd = jax.ShapeDtypeStruct((n, _C), jnp.float32)
    return pl.pallas_call(
        _qkv_body,
        grid=grid,
        in_specs=[row_spec, row_spec] + [w_spec] * 6,
        out_specs=[row_spec] * 6,
        out_shape=[out_sd] * 6,
    )(lf, rf, Wq1, Wk1, Wv1, Wq2, Wk2, Wv2)


def _build_grid(coors):
    lin = coors[:, 0] * _W + coors[:, 1]
    grid = jnp.full((_H * _W,), -1, dtype=jnp.int32)
    return grid.at[lin].max(jnp.arange(coors.shape[0], dtype=jnp.int32))


def _side(q_src, kv_src, q_coors, kv_grid, Wq, Wk, Wv, pos, q_map, k_map, v_map):
    del q_src, kv_src, Wq, Wk, Wv
    keys = []
    vals = []
    for i in range(9):
        shifted = q_coors + jnp.asarray(_INDEX_SHIFT[i])[None, :]
        valid = ((shifted[:, 0] >= 0) & (shifted[:, 0] < _H)
                 & (shifted[:, 1] >= 0) & (shifted[:, 1] < _W))
        lin = jnp.clip(shifted[:, 0], 0, _H - 1) * _W + jnp.clip(shifted[:, 1], 0, _W - 1)
        sel = jnp.where(valid, kv_grid[lin], -1)
        cond = (sel >= 0)[:, None]
        safe = jnp.clip(sel, 0, k_map.shape[0] - 1)
        tv = jnp.take(v_map, safe, axis=0) + pos[i][None, :]
        tk = jnp.take(k_map, safe, axis=0)
        vals.append(jnp.where(cond, tv, 0.0))
        keys.append(jnp.where(cond, tk, 0.0))
    Kt = jnp.stack(keys, axis=1)
    Vt = jnp.stack(vals, axis=1)
    logits = jnp.einsum('nc,nkc->nk', q_map, Kt) / jnp.sqrt(jnp.float32(_C))
    attn = jax.nn.softmax(logits, axis=-1)
    return jnp.einsum('nk,nkc->nc', attn, Vt)


def _canvas(out_rows, q_grid):
    table = jnp.concatenate([out_rows, jnp.zeros((1, _C), jnp.float32)], axis=0)
    idx = jnp.where(q_grid < 0, out_rows.shape[0], q_grid)
    canvasT = jnp.take(table, idx, axis=0)
    return canvasT.T.reshape(1, _C, _H, _W)


def kernel(li_bev_feats, li_bev_coors, ra_bev_feats, ra_bev_coors,
           pos_embedding, Wq1, Wk1, Wv1, Wq2, Wk2, Wv2):
    lf, lc = li_bev_feats[0], li_bev_coors[0]
    rf, rc = ra_bev_feats[0], ra_bev_coors[0]
    q1, k1, v1, q2, k2, v2 = _qkv(lf, rf, Wq1, Wk1, Wv1, Wq2, Wk2, Wv2)
    grid_li = _build_grid(lc)
    grid_ra = _build_grid(rc)
    out1 = _side(lf, rf, lc, grid_ra, Wq1, Wk1, Wv1, pos_embedding, q1, k1, v1)
    out2 = _side(rf, lf, rc, grid_li, Wq2, Wk2, Wv2, pos_embedding, q2, k2, v2)
    return _canvas(out1, grid_li), _canvas(out2, grid_ra)
